# baseline (speedup 1.0000x reference)
import functools

import jax
import jax.numpy as jnp
from jax import lax
from jax.experimental import pallas as pl
from jax.experimental.pallas import tpu as pltpu
from jax.experimental.pallas import tpu_sc as plsc

_DIM = 64
_BATCH = 16384
_NROW = 1_000_000
_L = 16

_NC = 2
_NS = 16
_NW = _NC * _NS
_NFULL = _NROW // 128
_TAILJ = _NFULL
_TAIL0 = _NFULL * 128
_RPW = 245
_SPAN = 4
_NF = 66
_CAP = 688
_SCAP = 32
_OUTR = _BATCH + 3 * _SCAP

_mesh = plsc.VectorSubcoreMesh(core_axis_name="c", subcore_axis_name="s")


@functools.partial(
    pl.kernel,
    mesh=_mesh,
    out_type=jax.ShapeDtypeStruct((_OUTR, 128), jnp.float32),
    scratch_types=[
        pltpu.VMEM((1024,), jnp.int32),
        pltpu.VMEM((_CAP + 32,), jnp.int32),
        pltpu.VMEM((_CAP + 32,), jnp.int32),
        pltpu.VMEM((272,), jnp.int32),
        pltpu.VMEM((272,), jnp.int32),
        pltpu.VMEM((64, _SPAN * 128), jnp.float32),
        pltpu.VMEM((64, _SPAN * 128), jnp.float32),
        pltpu.VMEM((64, _SPAN * 128), jnp.float32),
        pltpu.VMEM((64, 64), jnp.float32),
        pltpu.VMEM((3, _SCAP, 128), jnp.float32),
        pltpu.VMEM((1, _SCAP), jnp.int32),
        pltpu.VMEM((1, _SCAP), jnp.int32),
        pltpu.VMEM((1, _SCAP), jnp.int32),
        pltpu.SemaphoreType.DMA,
        pltpu.SemaphoreType.DMA,
        pltpu.SemaphoreType.DMA,
        pltpu.SemaphoreType.DMA,
        pltpu.SemaphoreType.DMA,
        pltpu.SemaphoreType.DMA,
    ],
    compiler_params=pltpu.CompilerParams(needs_layout_passes=False),
)
def _scan_kernel(idx_hbm, tab_hbm, tail_hbm, out_hbm, idxc_v, ki_v, kb_v,
                 mi_v, mb_v, blka_v, blkb_v, blkc_v, tail_v, stage_v, ob0_v,
                 ob1_v, ob2_v, sema, semb, semc, sems0, sems1, sems2):
  wid = lax.axis_index("s") * _NC + lax.axis_index("c")
  lo = wid * _RPW
  hi = lo + _RPW
  lane = lax.iota(jnp.int32, _L)

  def filt_chunk(ch, pos):
    pltpu.sync_copy(idx_hbm.at[pl.ds(ch * 1024, 1024)], idxc_v)

    def filt_group(g, pos):
      iv = idxc_v[pl.ds(g * _L, _L)]
      jv = iv >> 7
      m = (jv >= lo) & (jv < hi)
      cum = plsc.cumsum(m.astype(jnp.int32))
      dst = jnp.where(m, pos + cum - 1, _CAP + 24)
      plsc.store_scatter(ki_v, [dst], iv)
      bv = lane + (ch * 1024 + g * _L)
      plsc.store_scatter(kb_v, [dst], bv)
      return jnp.minimum(pos + cum[_L - 1], _CAP - _L)

    return lax.fori_loop(0, 1024 // _L, filt_group, pos)

  nkept = lax.fori_loop(0, _BATCH // 1024, filt_chunk, jnp.int32(0))
  ngrp = (nkept + _L - 1) >> 4

  obs = (ob0_v, ob1_v, ob2_v)
  for r in range(3):
    for g in range(_SCAP // _L):
      obs[r][0, pl.ds(g * _L, _L)] = jnp.full((_L,), _BATCH + r * _SCAP,
                                              jnp.int32) + lane + g * _L

  def match_range(jlo, jhi):
    def mgroup(t, mpos):
      kv = ki_v[pl.ds(t * _L, _L)]
      bv = kb_v[pl.ds(t * _L, _L)]
      jv = kv >> 7
      m = (jv >= jlo) & (jv < jhi) & ((lane + t * _L) < nkept)
      cum = plsc.cumsum(m.astype(jnp.int32))
      dst = jnp.where(m, mpos + cum - 1, 264)
      plsc.store_scatter(mi_v, [dst], kv)
      plsc.store_scatter(mb_v, [dst], bv)
      return jnp.minimum(mpos + cum[_L - 1], 256)

    return lax.fori_loop(0, ngrp, mgroup, jnp.int32(0))

  def extract(blk_ref, r, col_of, mpos, slot):
    def egroup(u, slot):
      v16i = mi_v[pl.ds(u * _L, _L)]
      v16b = mb_v[pl.ds(u * _L, _L)]
      for l in range(_L):
        active = (u * _L + l) < mpos

        @pl.when(active)
        def _(l=l, slot=slot):
          li = col_of(v16i[l])
          bpos = v16b[l]
          for g in range(_DIM // _L):
            vals = plsc.load_gather(
                blk_ref, [lane + g * _L, jnp.full((_L,), li, jnp.int32)])
            stage_v[r, slot, pl.ds(g * _L, _L)] = vals
          plsc.store_scatter(
              obs[r],
              [jnp.full((_L,), 0, jnp.int32),
               jnp.full((_L,), slot, jnp.int32)],
              jnp.full((_L,), bpos, jnp.int32))

        slot = jnp.where(active, jnp.minimum(slot + 1, _SCAP - 1), slot)
      return slot

    nu = (mpos + _L - 1) >> 4
    return lax.fori_loop(0, nu, egroup, slot)

  def fire_scatter(r, sem):
    pltpu.async_copy(stage_v.at[r], out_hbm.at[obs[r].at[0]], sem)

  def wait_scatter(r, sem):
    pltpu.make_async_copy(stage_v.at[r], out_hbm.at[obs[r].at[0]],
                          sem).wait()

  def start_fetch(f, buf, sem):
    j0c = jnp.minimum(lo + f * _SPAN, _NFULL - _SPAN)
    off = pl.multiple_of(j0c * 128, 128)
    pltpu.make_async_copy(tab_hbm.at[:, pl.ds(off, _SPAN * 128)], buf,
                          sem).start()

  def wait_fetch(buf, sem):
    pltpu.make_async_copy(tab_hbm.at[:, pl.ds(0, _SPAN * 128)], buf,
                          sem).wait()

  def process_fetch(f, blk, r, sems):
    j0 = lo + f * _SPAN
    j0c = jnp.minimum(j0, _NFULL - _SPAN)
    mpos = match_range(j0, j0 + _SPAN)
    base = j0c * 128
    extract(blk, r, lambda i: i - base, mpos, jnp.int32(0))
    fire_scatter(r, sems)

  bufs = (blka_v, blkb_v, blkc_v)
  fsems = (sema, semb, semc)
  ssems = (sems0, sems1, sems2)
  for r in range(3):
    fire_scatter(r, ssems[r])
    start_fetch(r, bufs[r], fsems[r])

  def tri(t, carry):
    for u in range(3):
      f = t * 3 + u
      wait_fetch(bufs[u], fsems[u])
      wait_scatter(u, ssems[u])
      process_fetch(f, bufs[u], u, ssems[u])

      @pl.when(f + 3 < _NF)
      def _(u=u, f=f):
        start_fetch(f + 3, bufs[u], fsems[u])

    return carry

  lax.fori_loop(0, _NF // 3, tri, jnp.int32(0))

  pltpu.sync_copy(tail_hbm, tail_v)
  mpos = match_range(jnp.int32(_TAILJ), jnp.int32(_TAILJ + 1))
  wait_scatter(0, sems0)
  extract(tail_v, 0, lambda i: i - _TAIL0, mpos, jnp.int32(0))
  fire_scatter(0, sems0)
  wait_scatter(0, sems0)
  wait_scatter(1, sems1)
  wait_scatter(2, sems2)


def kernel(batchInds, physiologicalProfile):
  tab_t = physiologicalProfile.T
  tail = physiologicalProfile[_TAIL0:, :].T
  out2 = _scan_kernel(batchInds, tab_t, tail)
  return out2[:_BATCH, :_DIM]
